# Initial kernel scaffold; baseline (speedup 1.0000x reference)
import jax
import jax.numpy as jnp
from jax.experimental import pallas as pl


def kernel(z, edge_index, _lambda, gt_edge, s_mask):
    raise NotImplementedError("write your pallas kernel here")


# speedup vs baseline: 1.9553x; 1.9553x over previous
import functools

import jax
import jax.numpy as jnp
from jax import lax
from jax.experimental import pallas as pl
from jax.experimental.pallas import tpu as pltpu
from jax.experimental.pallas import tpu_sc as plsc

L = 16
NC = 2
NS = 16
NW = NC * NS

_GDN = lax.GatherDimensionNumbers(
    offset_dims=(), collapsed_slice_dims=(0,), start_index_map=(0,))


def _perm(x, idx):
    return lax.gather(x, idx[:, None], _GDN, (1,),
                      mode=lax.GatherScatterMode.PROMISE_IN_BOUNDS)


def _spcl_sc(z, src, dst, gt, sm, lam):
    E = src.shape[0]
    D = z.shape[1]
    epw = E // NW
    B = 80
    nchunks = epw // B
    gpc = B // L

    mesh = plsc.VectorSubcoreMesh(core_axis_name="c", subcore_axis_name="s")

    @functools.partial(
        pl.kernel,
        mesh=mesh,
        out_type=jax.ShapeDtypeStruct((NW, L), jnp.float32),
        scratch_types=[
            pltpu.VMEM((B,), jnp.int32),
            pltpu.VMEM((B,), jnp.int32),
            pltpu.VMEM((B, D), jnp.float32),
            pltpu.VMEM((B, D), jnp.float32),
            pltpu.VMEM((B,), jnp.float32),
            pltpu.VMEM((B,), jnp.float32),
            pltpu.VMEM((L,), jnp.float32),
            pltpu.VMEM((L,), jnp.float32),
            pltpu.SemaphoreType.DMA,
            pltpu.SemaphoreType.DMA,
        ],
    )
    def k(z_h, src_h, dst_h, gt_h, sm_h, lam_h, out_h,
          sidx, didx, srows, drows, gtv, wv, lamv, outv, sem1, sem2):
        wid = lax.axis_index("c") * NS + lax.axis_index("s")
        pltpu.sync_copy(lam_h, lamv)
        lam_vec = lamv[...]
        nfc = D // L
        lane = lax.iota(jnp.int32, L)
        folds = [lane ^ w for w in (8, 4, 2, 1)]

        def chunk_body(ci, tot):
            base = pl.multiple_of(wid * epw + ci * B, 8)
            pltpu.sync_copy(src_h.at[pl.ds(base, B)], sidx)
            pltpu.sync_copy(dst_h.at[pl.ds(base, B)], didx)
            pltpu.sync_copy(gt_h.at[pl.ds(base, B)], gtv)
            pltpu.sync_copy(sm_h.at[pl.ds(base, B)], wv)
            cp1 = pltpu.async_copy(z_h.at[sidx], srows, sem1)
            cp2 = pltpu.async_copy(z_h.at[didx], drows, sem2)
            cp1.wait()
            cp2.wait()

            def group_body(g, tot):
                dotv = jnp.zeros((L,), jnp.float32)
                for e in range(L):
                    row = g * L + e
                    prods = [srows[row, pl.ds(c * L, L)]
                             * drows[row, pl.ds(c * L, L)]
                             for c in range(nfc)]
                    n = nfc
                    while n > 1:
                        prods = [prods[2 * i] + prods[2 * i + 1]
                                 for i in range(n // 2)] + prods[n & ~1:]
                        n = (n + 1) // 2
                    h = prods[0]
                    for f in folds:
                        h = h + _perm(h, f)
                    dotv = jnp.where(lane == e, h, dotv)
                p = 1.0 / (1.0 + jnp.exp(-dotv))
                diff = p - gtv[pl.ds(g * L, L)]
                return tot + wv[pl.ds(g * L, L)] * (diff * diff - lam_vec)

            return lax.fori_loop(0, gpc, group_body, tot)

        tot = lax.fori_loop(0, nchunks, chunk_body,
                            jnp.zeros((L,), jnp.float32))
        outv[...] = tot
        pltpu.sync_copy(outv, out_h.at[wid])

    return k(z, src, dst, gt, sm, lam)


def kernel(z, edge_index, _lambda, gt_edge, s_mask):
    src = edge_index[0].astype(jnp.int32)
    dst = edge_index[1].astype(jnp.int32)
    lam = jnp.full((L,), _lambda, jnp.float32)
    parts = _spcl_sc(z, src, dst,
                     gt_edge.astype(jnp.float32),
                     s_mask.astype(jnp.float32), lam)
    return jnp.sum(parts)


# speedup vs baseline: 7.2984x; 3.7326x over previous
import functools

import jax
import jax.numpy as jnp
from jax import lax
from jax.experimental import pallas as pl
from jax.experimental.pallas import tpu as pltpu
from jax.experimental.pallas import tpu_sc as plsc

L = 16
NC = 2
NS = 16
NW = NC * NS

_GDN = lax.GatherDimensionNumbers(
    offset_dims=(), collapsed_slice_dims=(0,), start_index_map=(0,))


def _perm(x, idx):
    return lax.gather(x, idx[:, None], _GDN, (1,),
                      mode=lax.GatherScatterMode.PROMISE_IN_BOUNDS)


def _spcl_sc(z, src, dst, gt, sm, lam):
    E = src.shape[0]
    D = z.shape[1]
    epw = E // NW
    B = 80
    nchunks = epw // B
    gpc = B // L
    nfc = D // L

    mesh = plsc.VectorSubcoreMesh(core_axis_name="c", subcore_axis_name="s")

    @functools.partial(
        pl.kernel,
        mesh=mesh,
        out_type=jax.ShapeDtypeStruct((NW, L), jnp.float32),
        scratch_types=[
            pltpu.VMEM((2, B), jnp.int32),
            pltpu.VMEM((2, B), jnp.int32),
            pltpu.VMEM((2, B, D), jnp.float32),
            pltpu.VMEM((2, B, D), jnp.float32),
            pltpu.VMEM((2, B), jnp.float32),
            pltpu.VMEM((2, B), jnp.float32),
            pltpu.VMEM((L,), jnp.float32),
            pltpu.VMEM((L,), jnp.float32),
            pltpu.SemaphoreType.DMA,
            pltpu.SemaphoreType.DMA,
        ],
    )
    def k(z_h, src_h, dst_h, gt_h, sm_h, lam_h, out_h,
          sidx, didx, srows, drows, gtv, wv, lamv, outv, sem_i, sem_r):
        wid = lax.axis_index("c") * NS + lax.axis_index("s")
        pltpu.sync_copy(lam_h, lamv)
        lam_vec = lamv[...]
        lane = lax.iota(jnp.int32, L)
        perms = {w: lane ^ w for w in (8, 4, 2, 1)}

        def issue_idx(ci, buf):
            base = pl.multiple_of(wid * epw + ci * B, 8)
            pltpu.async_copy(src_h.at[pl.ds(base, B)], sidx.at[buf], sem_i)
            pltpu.async_copy(dst_h.at[pl.ds(base, B)], didx.at[buf], sem_i)
            pltpu.async_copy(gt_h.at[pl.ds(base, B)], gtv.at[buf], sem_i)
            pltpu.async_copy(sm_h.at[pl.ds(base, B)], wv.at[buf], sem_i)

        def wait_idx(buf):
            pltpu.make_async_copy(src_h.at[pl.ds(0, B)], sidx.at[buf],
                                  sem_i).wait()
            pltpu.make_async_copy(dst_h.at[pl.ds(0, B)], didx.at[buf],
                                  sem_i).wait()
            pltpu.make_async_copy(gt_h.at[pl.ds(0, B)], gtv.at[buf],
                                  sem_i).wait()
            pltpu.make_async_copy(sm_h.at[pl.ds(0, B)], wv.at[buf],
                                  sem_i).wait()

        def issue_rows(buf):
            pltpu.async_copy(z_h.at[sidx.at[buf]], srows.at[buf], sem_r)
            pltpu.async_copy(z_h.at[didx.at[buf]], drows.at[buf], sem_r)

        def wait_rows(buf):
            pltpu.make_async_copy(z_h.at[sidx.at[buf]], srows.at[buf],
                                  sem_r).wait()
            pltpu.make_async_copy(z_h.at[didx.at[buf]], drows.at[buf],
                                  sem_r).wait()

        issue_idx(0, 0)
        wait_idx(0)
        issue_rows(0)
        issue_idx(1, 1)

        def chunk_body(c, tot):
            buf = lax.rem(c, 2)
            nbuf = 1 - buf
            wait_rows(buf)

            @pl.when(c + 1 < nchunks)
            def _():
                wait_idx(nbuf)
                issue_rows(nbuf)

            @pl.when(c + 2 < nchunks)
            def _():
                issue_idx(c + 2, buf)

            def edge_body(e, acc):
                ew = lax.rem(e, L)
                gb = pl.multiple_of(e - ew, 8)
                prods = [srows[buf, e, pl.ds(f * L, L)]
                         * drows[buf, e, pl.ds(f * L, L)]
                         for f in range(nfc)]
                n = nfc
                while n > 1:
                    prods = [prods[2 * i] + prods[2 * i + 1]
                             for i in range(n // 2)] + prods[n & ~1:]
                    n = (n + 1) // 2
                h = prods[0]
                for w in (8, 4, 2, 1):
                    h = h + _perm(h, perms[w])
                p = 1.0 / (1.0 + jnp.exp(-h))
                diff = p - gtv[buf, pl.ds(gb, L)]
                cont = wv[buf, pl.ds(gb, L)] * (diff * diff - lam_vec)
                return acc + jnp.where(lane == ew, cont, 0.0)

            return plsc.parallel_loop(0, B, unroll=4, carry=tot)(edge_body)

        tot = lax.fori_loop(0, nchunks, chunk_body,
                            jnp.zeros((L,), jnp.float32))
        outv[...] = tot
        pltpu.sync_copy(outv, out_h.at[wid])

    return k(z, src, dst, gt, sm, lam)


def kernel(z, edge_index, _lambda, gt_edge, s_mask):
    src = edge_index[0].astype(jnp.int32)
    dst = edge_index[1].astype(jnp.int32)
    lam = jnp.full((L,), _lambda, jnp.float32)
    parts = _spcl_sc(z, src, dst,
                     gt_edge.astype(jnp.float32),
                     s_mask.astype(jnp.float32), lam)
    return jnp.sum(parts)


# speedup vs baseline: 11.6117x; 1.5910x over previous
import functools

import jax
import jax.numpy as jnp
from jax import lax
from jax.experimental import pallas as pl
from jax.experimental.pallas import tpu as pltpu
from jax.experimental.pallas import tpu_sc as plsc

L = 16
NC = 2
NS = 16
NW = NC * NS

_GDN = lax.GatherDimensionNumbers(
    offset_dims=(), collapsed_slice_dims=(0,), start_index_map=(0,))


def _perm(x, idx):
    return lax.gather(x, idx[:, None], _GDN, (1,),
                      mode=lax.GatherScatterMode.PROMISE_IN_BOUNDS)


def _spcl_sc(zw, src, dst, gt, sm, lam):
    E = src.shape[0]
    Dw = zw.shape[1]
    nwc = Dw // L
    epw = E // NW
    B = 400
    SG = 80
    nchunks = epw // B

    mesh = plsc.VectorSubcoreMesh(core_axis_name="c", subcore_axis_name="s")

    @functools.partial(
        pl.kernel,
        mesh=mesh,
        out_type=jax.ShapeDtypeStruct((NW, L), jnp.float32),
        compiler_params=pltpu.CompilerParams(needs_layout_passes=False, use_tc_tiling_on_sc=False),
        scratch_types=[
            pltpu.VMEM((2 * B,), jnp.int32),
            pltpu.VMEM((2 * B,), jnp.int32),
            pltpu.VMEM((2 * B, Dw), jnp.float32),
            pltpu.VMEM((2 * B, Dw), jnp.float32),
            pltpu.VMEM((2 * B,), jnp.float32),
            pltpu.VMEM((2 * B,), jnp.float32),
            pltpu.VMEM((L,), jnp.float32),
            pltpu.VMEM((L,), jnp.float32),
            pltpu.SemaphoreType.DMA,
            pltpu.SemaphoreType.DMA,
        ],
    )
    def k(z_h, src_h, dst_h, gt_h, sm_h, lam_h, out_h,
          sidx, didx, srows, drows, gtv, wv, lamv, outv, sem_i, sem_r):
        wid = lax.axis_index("c") * NS + lax.axis_index("s")
        pltpu.sync_copy(lam_h, lamv)
        lam_vec = lamv[...]
        lane = lax.iota(jnp.int32, L)
        perms = {w: lane ^ w for w in (8, 4, 2, 1)}

        def issue_idx(ci, buf):
            base = pl.multiple_of(wid * epw + ci * B, 8)
            bo = pl.multiple_of(buf * B, 8)
            pltpu.async_copy(src_h.at[pl.ds(base, B)],
                             sidx.at[pl.ds(bo, B)], sem_i)
            pltpu.async_copy(dst_h.at[pl.ds(base, B)],
                             didx.at[pl.ds(bo, B)], sem_i)
            pltpu.async_copy(gt_h.at[pl.ds(base, B)],
                             gtv.at[pl.ds(bo, B)], sem_i)
            pltpu.async_copy(sm_h.at[pl.ds(base, B)],
                             wv.at[pl.ds(bo, B)], sem_i)

        def wait_idx(buf):
            bo = pl.multiple_of(buf * B, 8)
            pltpu.make_async_copy(src_h.at[pl.ds(0, B)],
                                  sidx.at[pl.ds(bo, B)], sem_i).wait()
            pltpu.make_async_copy(dst_h.at[pl.ds(0, B)],
                                  didx.at[pl.ds(bo, B)], sem_i).wait()
            pltpu.make_async_copy(gt_h.at[pl.ds(0, B)],
                                  gtv.at[pl.ds(bo, B)], sem_i).wait()
            pltpu.make_async_copy(sm_h.at[pl.ds(0, B)],
                                  wv.at[pl.ds(bo, B)], sem_i).wait()

        def issue_rows(buf):
            for j in range(B // SG):
                s = pl.ds(pl.multiple_of(buf * B + j * SG, 8), SG)
                pltpu.async_copy(z_h.at[sidx.at[s]], srows.at[s], sem_r)
                pltpu.async_copy(z_h.at[didx.at[s]], drows.at[s], sem_r)

        def wait_rows(buf):
            for j in range(B // SG):
                s = pl.ds(pl.multiple_of(buf * B + j * SG, 8), SG)
                pltpu.make_async_copy(z_h.at[sidx.at[s]], srows.at[s],
                                      sem_r).wait()
                pltpu.make_async_copy(z_h.at[didx.at[s]], drows.at[s],
                                      sem_r).wait()

        issue_idx(0, 0)
        wait_idx(0)
        issue_rows(0)
        issue_idx(1, 1)

        def chunk_body(c, tot):
            buf = lax.rem(c, 2)
            nbuf = 1 - buf
            wait_rows(buf)

            @pl.when(c + 1 < nchunks)
            def _():
                wait_idx(nbuf)
                issue_rows(nbuf)

            @pl.when(c + 2 < nchunks)
            def _():
                issue_idx(c + 2, buf)

            bo = pl.multiple_of(buf * B, 8)

            def edge_body(e, acc):
                ew = lax.rem(e, L)
                gb = pl.multiple_of(bo + e - ew, 8)
                row = bo + e
                ps = []
                for f in range(nwc):
                    a = plsc.bitcast(srows[row, pl.ds(f * L, L)],
                                     jnp.bfloat16)
                    b = plsc.bitcast(drows[row, pl.ds(f * L, L)],
                                     jnp.bfloat16)
                    ps.append(a * b)
                n = nwc
                while n > 1:
                    ps = [ps[2 * i] + ps[2 * i + 1]
                          for i in range(n // 2)] + ps[n & ~1:]
                    n = (n + 1) // 2
                ev, od = plsc.unpack(ps[0], format=plsc.PackFormat.INTERLEAVED,
                                     preferred_element_type=jnp.float32)
                h = ev + od
                for w in (8, 4, 2, 1):
                    h = h + _perm(h, perms[w])
                p = 1.0 / (1.0 + jnp.exp(-h))
                diff = p - gtv[pl.ds(gb, L)]
                cont = wv[pl.ds(gb, L)] * (diff * diff - lam_vec)
                return acc + jnp.where(lane == ew, cont, 0.0)

            return plsc.parallel_loop(0, B, unroll=4, carry=tot)(edge_body)

        tot = lax.fori_loop(0, nchunks, chunk_body,
                            jnp.zeros((L,), jnp.float32))
        outv[...] = tot
        pltpu.sync_copy(outv, out_h.at[wid])

    return k(zw, src, dst, gt, sm, lam)


def kernel(z, edge_index, _lambda, gt_edge, s_mask):
    n, d = z.shape
    zw = lax.bitcast_convert_type(
        z.astype(jnp.bfloat16).reshape(n, d // 2, 2), jnp.float32)
    src = edge_index[0].astype(jnp.int32)
    dst = edge_index[1].astype(jnp.int32)
    lam = jnp.full((L,), _lambda, jnp.float32)
    parts = _spcl_sc(zw, src, dst,
                     gt_edge.astype(jnp.float32),
                     s_mask.astype(jnp.float32), lam)
    return jnp.sum(parts)
